# baseline (speedup 1.0000x reference)
import functools

import jax
import jax.numpy as jnp
from jax import lax
from jax.experimental import pallas as pl
from jax.experimental.pallas import tpu as pltpu
from jax.experimental.pallas import tpu_sc as plsc

B = 16384
N = 1000000
J = 100000
K = 32
NC = 2
NS = 16
NW = NC * NS
BPW = B // NW
CHUNK = 128
NCHUNK = BPW // CHUNK

SLAB = 32768
WAVE = 512
NWAVE_MAX = SLAB // WAVE
CAP = 640
NGRP = CAP // CHUNK
SINK = B
UG_ROWS = B + CHUNK

_I16 = None


def _iota16():
    return jnp.arange(16, dtype=jnp.int32)


def _stream_kernel(users_hbm, Ut_hbm, ug_hbm,
                   uchunk, lr, li, wl_r, wl_p, extr, li2d, wave_buf,
                   sem_a, sem_b, sem_s):
    wid = lax.axis_index("s") * NC + lax.axis_index("c")
    lane = _iota16()

    def chunk_body(c, cnt):
        pltpu.sync_copy(users_hbm.at[pl.ds(c * 1024, 1024)], uchunk)

        def vec_body(v, cnt):
            uv = uchunk[pl.ds(v * 16, 16)]
            m = lax.shift_right_logical(uv, 15) == wid
            ival = c * 1024 + v * 16 + lane
            plsc.store_compressed(lr.at[pl.ds(cnt, 16)], uv, mask=m)
            plsc.store_compressed(li.at[pl.ds(cnt, 16)], ival, mask=m)
            npc = jnp.sum(m.astype(jnp.int32))
            return jnp.minimum(cnt + npc, CAP)

        return lax.fori_loop(0, 64, vec_body, cnt)

    cnt = lax.fori_loop(0, 16, chunk_body, jnp.int32(0))
    ngroups = (cnt + 15) // 16

    base = wid * SLAB
    nvalid = jnp.clip(N - base, 0, SLAB)
    nwaves = (nvalid + WAVE - 1) // WAVE

    pad_n = ((N + 127) // 128) * 128

    def wstart(w):
        s0 = jnp.minimum(base + w * WAVE, pad_n - WAVE)
        return pl.multiple_of(s0, 128)

    sems = [sem_a, sem_b]

    def fire(w, slot_sems):
        s0 = wstart(w)
        for slot in (0, 1):
            @pl.when((w % 2) == slot)
            def _():
                pltpu.async_copy(Ut_hbm.at[:, pl.ds(s0, WAVE)],
                                 wave_buf.at[slot], slot_sems[slot])

    def drain(w, slot_sems):
        s0 = wstart(w)
        for slot in (0, 1):
            @pl.when((w % 2) == slot)
            def _():
                pltpu.make_async_copy(Ut_hbm.at[:, pl.ds(s0, WAVE)],
                                      wave_buf.at[slot],
                                      slot_sems[slot]).wait()

    fire(jnp.int32(0), sems)

    def wave_body(w, carry):
        fire(w + 1, sems)
        drain(w, sems)
        s0 = wstart(w)
        slotv = jnp.full((16,), w % 2, dtype=jnp.int32)

        def cgrp(g, wcnt):
            rv = lr[pl.ds(g * 16, 16)]
            pos = g * 16 + lane
            m = (pos < cnt) & (rv >= s0) & (rv < s0 + WAVE)
            plsc.store_compressed(wl_r.at[pl.ds(wcnt, 16)], rv - s0, mask=m)
            plsc.store_compressed(wl_p.at[pl.ds(wcnt, 16)], pos, mask=m)
            return wcnt + jnp.sum(m.astype(jnp.int32))

        wcnt = lax.fori_loop(0, ngroups, cgrp, jnp.int32(0))

        def egrp(h, carry):
            roff = wl_r[pl.ds(h * 16, 16)]
            p = wl_p[pl.ds(h * 16, 16)]
            am = (h * 16 + lane) < wcnt
            for k in range(K):
                kf = jnp.full((16,), k, dtype=jnp.int32)
                vals = plsc.load_gather(wave_buf, [slotv, kf, roff], mask=am)
                plsc.store_scatter(extr, [p, kf], vals, mask=am)
            return carry

        lax.fori_loop(0, (wcnt + 15) // 16, egrp, 0)
        return carry

    lax.fori_loop(0, nwaves, wave_body, 0)
    drain(nwaves, sems)

    for t in range(NGRP):
        for u in range(8):
            pos = t * 128 + u * 16 + lane
            vals = li[pl.ds(t * 128 + u * 16, 16)]
            sink = SINK + u * 16 + lane
            li2d[t, pl.ds(u * 16, 16)] = jnp.where(pos < cnt, vals, sink)
    handles = []
    for t in range(NGRP):
        handles.append(
            pltpu.async_copy(extr.at[pl.ds(t * 128, 128)],
                             ug_hbm.at[li2d.at[t]], sem_s))
    for h in handles:
        h.wait()


@jax.jit
def _stage_u(users, Ut):
    mesh = plsc.VectorSubcoreMesh(core_axis_name="c", subcore_axis_name="s")
    f = functools.partial(
        pl.kernel,
        mesh=mesh,
        out_type=jax.ShapeDtypeStruct((UG_ROWS, CHUNK), jnp.float32),
        scratch_types=[
            pltpu.VMEM((1024,), jnp.int32),
            pltpu.VMEM((CAP + 16,), jnp.int32),
            pltpu.VMEM((CAP + 16,), jnp.int32),
            pltpu.VMEM((80,), jnp.int32),
            pltpu.VMEM((80,), jnp.int32),
            pltpu.VMEM((CAP, CHUNK), jnp.float32),
            pltpu.VMEM((NGRP, CHUNK), jnp.int32),
            pltpu.VMEM((2, K, WAVE), jnp.float32),
            pltpu.SemaphoreType.DMA,
            pltpu.SemaphoreType.DMA,
            pltpu.SemaphoreType.DMA,
        ],
        compiler_params=pltpu.CompilerParams(
            needs_layout_passes=False, use_tc_tiling_on_sc=True
        ),
    )(_stream_kernel)
    return f(users, Ut)


def _dot_kernel(users_hbm, jokes_hbm, ug_hbm, Vw_hbm, a_hbm, b_hbm, g_hbm,
                out_hbm,
                idx_u, idx_j, div_j, rem_j, u_buf, v_buf, a_v, b_v, g_v,
                out_v, *sems):
    wid = lax.axis_index("s") * NC + lax.axis_index("c")
    base = wid * BPW
    lane = _iota16()

    for j in range(NCHUNK):
        pltpu.sync_copy(users_hbm.at[pl.ds(base + j * CHUNK, CHUNK)],
                        idx_u.at[j])
        pltpu.sync_copy(jokes_hbm.at[pl.ds(base + j * CHUNK, CHUNK)],
                        idx_j.at[j])
    pltpu.sync_copy(g_hbm, g_v)

    def split_body(i, carry):
        j = i // 8
        s = (i % 8) * 16
        rj = idx_j[j, pl.ds(s, 16)]
        div_j[j, pl.ds(s, 16)] = lax.shift_right_logical(rj, 2)
        rem_j[j, pl.ds(s, 16)] = lax.shift_left(rj & 3, 5)
        return carry

    lax.fori_loop(0, NCHUNK * 8, split_body, 0)

    ab_handles = []
    for j in range(NCHUNK):
        ab_handles.append(
            pltpu.async_copy(a_hbm.at[idx_u.at[j]], a_v.at[j], sems[NCHUNK]))
        ab_handles.append(
            pltpu.async_copy(b_hbm.at[idx_j.at[j]], b_v.at[j], sems[NCHUNK]))

    handles = {}

    def fire(j):
        slot = j % 2
        hu = pltpu.async_copy(
            ug_hbm.at[pl.ds(base + j * CHUNK, CHUNK)], u_buf.at[slot],
            sems[j])
        hv = pltpu.async_copy(
            Vw_hbm.at[div_j.at[j]], v_buf.at[slot], sems[j])
        handles[j] = [hu, hv]

    fire(0)
    fire(1)
    for h in ab_handles:
        h.wait()

    gvec = g_v[...]

    for j in range(NCHUNK):
        slot = j % 2
        for h in handles[j]:
            h.wait()
        sf = jnp.full((16,), slot, dtype=jnp.int32)

        def group_body(grp, carry):
            s = grp * 16
            row = lane + s
            cv = rem_j[j, pl.ds(s, 16)]
            acc = jnp.zeros((16,), dtype=jnp.float32)
            for k in range(K):
                kf = jnp.full((16,), k, dtype=jnp.int32)
                uk = plsc.load_gather(u_buf, [sf, row, kf])
                vk = plsc.load_gather(v_buf, [sf, row, cv + k])
                acc = acc + uk * vk
            ab = a_v[j, pl.ds(s, 16)] + b_v[j, pl.ds(s, 16)]
            out_v[pl.ds(j * CHUNK + s, 16)] = acc + ab + gvec
            return carry

        lax.fori_loop(0, CHUNK // 16, group_body, 0)
        if j + 2 < NCHUNK:
            fire(j + 2)

    pltpu.sync_copy(out_v, out_hbm.at[pl.ds(base, BPW)])


@jax.jit
def _dot(users, jokes, ug, Vw, a_flat, b_flat, g16):
    mesh = plsc.VectorSubcoreMesh(core_axis_name="c", subcore_axis_name="s")
    f = functools.partial(
        pl.kernel,
        mesh=mesh,
        out_type=jax.ShapeDtypeStruct((B,), jnp.float32),
        scratch_types=[
            pltpu.VMEM((NCHUNK, CHUNK), jnp.int32),
            pltpu.VMEM((NCHUNK, CHUNK), jnp.int32),
            pltpu.VMEM((NCHUNK, CHUNK), jnp.int32),
            pltpu.VMEM((NCHUNK, CHUNK), jnp.int32),
            pltpu.VMEM((2, CHUNK, CHUNK), jnp.float32),
            pltpu.VMEM((2, CHUNK, CHUNK), jnp.float32),
            pltpu.VMEM((NCHUNK, CHUNK), jnp.float32),
            pltpu.VMEM((NCHUNK, CHUNK), jnp.float32),
            pltpu.VMEM((16,), jnp.float32),
            pltpu.VMEM((BPW,), jnp.float32),
        ] + [pltpu.SemaphoreType.DMA] * (NCHUNK + 1),
        compiler_params=pltpu.CompilerParams(
            needs_layout_passes=False, use_tc_tiling_on_sc=False
        ),
    )(_dot_kernel)
    return f(users, jokes, ug, Vw, a_flat, b_flat, g16)


def kernel(users, jokes, U, V, a, b, g):
    users = users.astype(jnp.int32)
    jokes = jokes.astype(jnp.int32)
    ug = _stage_u(users, U.T)
    Vw = V.reshape(V.shape[0] * K // 128, 128)
    g16 = jnp.broadcast_to(g.astype(jnp.float32), (16,))
    return _dot(users, jokes, ug, Vw, a.reshape(-1), b.reshape(-1), g16)


# speedup vs baseline: 1.1323x; 1.1323x over previous
import functools

import jax
import jax.numpy as jnp
from jax import lax
from jax.experimental import pallas as pl
from jax.experimental.pallas import tpu as pltpu
from jax.experimental.pallas import tpu_sc as plsc

B = 16384
N = 1000000
J = 100000
K = 32
NC = 2
NS = 16
NW = NC * NS
BPW = B // NW
CHUNK = 128
NCHUNK = BPW // CHUNK

HSLAB = 16384
WAVE = 1024
CAP = 384
NGRP = CAP // CHUNK
HOFF = 512
SINK = B
UG_ROWS = B + CHUNK

_I16 = None


def _iota16():
    return jnp.arange(16, dtype=jnp.int32)


def _stream_kernel(users_hbm, Ut_hbm, ug_hbm,
                   uchunk, lr, li, wl_r, wl_p, extr, li2d, wave_buf,
                   sem_a, sem_b, sem_s):
    wid = lax.axis_index("s") * NC + lax.axis_index("c")
    lane = _iota16()
    pad_n = ((N + 127) // 128) * 128
    sems = [sem_a, sem_b]

    def wstart(half, w):
        base = (wid * 2 + half) * HSLAB
        s0 = jnp.minimum(base + w * WAVE, pad_n - WAVE)
        return pl.multiple_of(s0, 128)

    def fire(half, w):
        s0 = wstart(half, w)
        for slot in (0, 1):
            @pl.when((w % 2) == slot)
            def _():
                pltpu.async_copy(Ut_hbm.at[:, pl.ds(s0, WAVE)],
                                 wave_buf.at[slot], sems[slot])

    def drain(half, w):
        s0 = wstart(half, w)
        for slot in (0, 1):
            @pl.when((w % 2) == slot)
            def _():
                pltpu.make_async_copy(Ut_hbm.at[:, pl.ds(s0, WAVE)],
                                      wave_buf.at[slot], sems[slot]).wait()

    fire(0, jnp.int32(0))

    def chunk_body(c, cnts):
        pltpu.sync_copy(users_hbm.at[pl.ds(c * 1024, 1024)], uchunk)

        def vec_body(v, cnts):
            c0, c1 = cnts
            uv = uchunk[pl.ds(v * 16, 16)]
            hid = lax.shift_right_logical(uv, 14)
            ival = c * 1024 + v * 16 + lane
            m0 = hid == wid * 2
            m1 = hid == wid * 2 + 1
            plsc.store_compressed(lr.at[pl.ds(c0, 16)], uv, mask=m0)
            plsc.store_compressed(li.at[pl.ds(c0, 16)], ival, mask=m0)
            plsc.store_compressed(lr.at[pl.ds(HOFF + c1, 16)], uv, mask=m1)
            plsc.store_compressed(li.at[pl.ds(HOFF + c1, 16)], ival, mask=m1)
            c0 = jnp.minimum(c0 + jnp.sum(m0.astype(jnp.int32)), CAP)
            c1 = jnp.minimum(c1 + jnp.sum(m1.astype(jnp.int32)), CAP)
            return c0, c1

        return lax.fori_loop(0, 64, vec_body, cnts)

    cnts = lax.fori_loop(0, 16, chunk_body,
                         (jnp.int32(0), jnp.int32(0)))

    for half in (0, 1):
        cnt = cnts[half]
        ngroups = (cnt + 15) // 16
        base = (wid * 2 + half) * HSLAB
        nvalid = jnp.clip(N - base, 0, HSLAB)
        nwaves = (nvalid + WAVE - 1) // WAVE

        if half == 1:
            fire(1, jnp.int32(0))

        def wave_body(w, carry):
            fire(half, w + 1)
            drain(half, w)
            s0 = wstart(half, w)
            slotv = jnp.full((16,), w % 2, dtype=jnp.int32)

            def cgrp(g, wcnt):
                rv = lr[pl.ds(half * HOFF + g * 16, 16)]
                pos = g * 16 + lane
                m = (pos < cnt) & (rv >= s0) & (rv < s0 + WAVE)
                plsc.store_compressed(wl_r.at[pl.ds(wcnt, 16)], rv - s0,
                                      mask=m)
                plsc.store_compressed(wl_p.at[pl.ds(wcnt, 16)], pos, mask=m)
                return wcnt + jnp.sum(m.astype(jnp.int32))

            wcnt = lax.fori_loop(0, ngroups, cgrp, jnp.int32(0))

            def egrp(h, carry):
                roff = wl_r[pl.ds(h * 16, 16)]
                p = wl_p[pl.ds(h * 16, 16)]
                am = (h * 16 + lane) < wcnt
                for k in range(K):
                    kf = jnp.full((16,), k, dtype=jnp.int32)
                    vals = plsc.load_gather(wave_buf, [slotv, kf, roff],
                                            mask=am)
                    plsc.store_scatter(extr, [p, kf], vals, mask=am)
                return carry

            lax.fori_loop(0, (wcnt + 15) // 16, egrp, 0)
            return carry

        lax.fori_loop(0, nwaves, wave_body, 0)
        drain(half, nwaves)

        for t in range(NGRP):
            for u in range(8):
                pos = t * 128 + u * 16 + lane
                vals = li[pl.ds(half * HOFF + t * 128 + u * 16, 16)]
                sink = SINK + u * 16 + lane
                li2d[t, pl.ds(u * 16, 16)] = jnp.where(pos < cnt, vals, sink)
        handles = []
        for t in range(NGRP):
            handles.append(
                pltpu.async_copy(extr.at[pl.ds(t * 128, 128)],
                                 ug_hbm.at[li2d.at[t]], sem_s))
        for h in handles:
            h.wait()


@jax.jit
def _stage_u(users, Ut):
    mesh = plsc.VectorSubcoreMesh(core_axis_name="c", subcore_axis_name="s")
    f = functools.partial(
        pl.kernel,
        mesh=mesh,
        out_type=jax.ShapeDtypeStruct((UG_ROWS, CHUNK), jnp.float32),
        scratch_types=[
            pltpu.VMEM((1024,), jnp.int32),
            pltpu.VMEM((2 * HOFF,), jnp.int32),
            pltpu.VMEM((2 * HOFF,), jnp.int32),
            pltpu.VMEM((96,), jnp.int32),
            pltpu.VMEM((96,), jnp.int32),
            pltpu.VMEM((CAP, CHUNK), jnp.float32),
            pltpu.VMEM((NGRP, CHUNK), jnp.int32),
            pltpu.VMEM((2, K, WAVE), jnp.float32),
            pltpu.SemaphoreType.DMA,
            pltpu.SemaphoreType.DMA,
            pltpu.SemaphoreType.DMA,
        ],
        compiler_params=pltpu.CompilerParams(
            needs_layout_passes=False, use_tc_tiling_on_sc=True
        ),
    )(_stream_kernel)
    return f(users, Ut)


def _dot_kernel(users_hbm, jokes_hbm, ug_hbm, Vt_hbm, a_hbm, b_hbm, g_hbm,
                out_hbm,
                idx_u, idx_j, u_buf, v_buf, a_v, b_v, g_v, out_v,
                *sems):
    wid = lax.axis_index("s") * NC + lax.axis_index("c")
    base = wid * BPW
    lane = _iota16()

    for j in range(NCHUNK):
        pltpu.sync_copy(users_hbm.at[pl.ds(base + j * CHUNK, CHUNK)],
                        idx_u.at[j])
        pltpu.sync_copy(jokes_hbm.at[pl.ds(base + j * CHUNK, CHUNK)],
                        idx_j.at[j])
    pltpu.sync_copy(g_hbm, g_v)

    ab_handles = []
    for j in range(NCHUNK):
        ab_handles.append(
            pltpu.async_copy(a_hbm.at[idx_u.at[j]], a_v.at[j], sems[NCHUNK]))
        ab_handles.append(
            pltpu.async_copy(b_hbm.at[idx_j.at[j]], b_v.at[j], sems[NCHUNK]))

    u_handles = {}

    def fire(j):
        u_handles[j] = pltpu.async_copy(
            ug_hbm.at[pl.ds(base + j * CHUNK, CHUNK)], u_buf.at[j], sems[j])

        def body(k, carry):
            pltpu.async_copy(Vt_hbm.at[k].at[idx_j.at[j]],
                             v_buf.at[j, k], sems[j])
            return carry
        lax.fori_loop(0, K, body, 0)

    def drain(j):
        u_handles[j].wait()

        def body(k, carry):
            pltpu.make_async_copy(Vt_hbm.at[k].at[idx_j.at[j]],
                                  v_buf.at[j, k], sems[j]).wait()
            return carry
        lax.fori_loop(0, K, body, 0)

    fire(0)
    fire(1)
    for h in ab_handles:
        h.wait()

    gvec = g_v[...]

    for j in range(NCHUNK):
        drain(j)
        jf = jnp.full((16,), j, dtype=jnp.int32)

        def group_body(grp, carry):
            s = grp * 16
            row = lane + s
            acc = jnp.zeros((16,), dtype=jnp.float32)
            for k in range(K):
                kf = jnp.full((16,), k, dtype=jnp.int32)
                uk = plsc.load_gather(u_buf, [jf, row, kf])
                acc = acc + uk * v_buf[j, k, pl.ds(s, 16)]
            ab = a_v[j, pl.ds(s, 16)] + b_v[j, pl.ds(s, 16)]
            out_v[pl.ds(j * CHUNK + s, 16)] = acc + ab + gvec
            return carry

        lax.fori_loop(0, CHUNK // 16, group_body, 0)
        if j + 2 < NCHUNK:
            fire(j + 2)

    pltpu.sync_copy(out_v, out_hbm.at[pl.ds(base, BPW)])


@jax.jit
def _dot(users, jokes, ug, Vt, a_flat, b_flat, g16):
    mesh = plsc.VectorSubcoreMesh(core_axis_name="c", subcore_axis_name="s")
    f = functools.partial(
        pl.kernel,
        mesh=mesh,
        out_type=jax.ShapeDtypeStruct((B,), jnp.float32),
        scratch_types=[
            pltpu.VMEM((NCHUNK, CHUNK), jnp.int32),
            pltpu.VMEM((NCHUNK, CHUNK), jnp.int32),
            pltpu.VMEM((NCHUNK, CHUNK, CHUNK), jnp.float32),
            pltpu.VMEM((NCHUNK, K, CHUNK), jnp.float32),
            pltpu.VMEM((NCHUNK, CHUNK), jnp.float32),
            pltpu.VMEM((NCHUNK, CHUNK), jnp.float32),
            pltpu.VMEM((16,), jnp.float32),
            pltpu.VMEM((BPW,), jnp.float32),
        ] + [pltpu.SemaphoreType.DMA] * (NCHUNK + 1),
        compiler_params=pltpu.CompilerParams(
            needs_layout_passes=False, use_tc_tiling_on_sc=False
        ),
    )(_dot_kernel)
    return f(users, jokes, ug, Vt, a_flat, b_flat, g16)


def kernel(users, jokes, U, V, a, b, g):
    users = users.astype(jnp.int32)
    jokes = jokes.astype(jnp.int32)
    ug = _stage_u(users, U.T)
    g16 = jnp.broadcast_to(g.astype(jnp.float32), (16,))
    return _dot(users, jokes, ug, V.T, a.reshape(-1), b.reshape(-1), g16)


# speedup vs baseline: 1.1380x; 1.0050x over previous
import functools

import jax
import jax.numpy as jnp
from jax import lax
from jax.experimental import pallas as pl
from jax.experimental.pallas import tpu as pltpu
from jax.experimental.pallas import tpu_sc as plsc

B = 16384
N = 1000000
J = 100000
K = 32
NC = 2
NS = 16
NW = NC * NS
BPW = B // NW
CHUNK = 128
NCHUNK = BPW // CHUNK

HSLAB = 16384
WAVE = 768
NSLOT = 3
CAP = 384
NGRP = CAP // CHUNK
HOFF = 512
SINK = B
UG_ROWS = B + CHUNK

_I16 = None


def _iota16():
    return jnp.arange(16, dtype=jnp.int32)


def _stream_kernel(users_hbm, Ut_hbm, ug_hbm,
                   uchunk, lr, li, wl_r, wl_p, extr, li2d, wave_buf,
                   sem_a, sem_b, sem_c, sem_s):
    wid = lax.axis_index("s") * NC + lax.axis_index("c")
    lane = _iota16()
    pad_n = ((N + 127) // 128) * 128
    sems = [sem_a, sem_b, sem_c]

    def wstart(half, w):
        base = (wid * 2 + half) * HSLAB
        s0 = jnp.minimum(base + w * WAVE, pad_n - WAVE)
        return pl.multiple_of(s0, 128)

    def fire(half, w):
        s0 = wstart(half, w)
        for slot in range(NSLOT):
            @pl.when((w % NSLOT) == slot)
            def _():
                pltpu.async_copy(Ut_hbm.at[:, pl.ds(s0, WAVE)],
                                 wave_buf.at[slot], sems[slot])

    def drain(half, w):
        s0 = wstart(half, w)
        for slot in range(NSLOT):
            @pl.when((w % NSLOT) == slot)
            def _():
                pltpu.make_async_copy(Ut_hbm.at[:, pl.ds(s0, WAVE)],
                                      wave_buf.at[slot], sems[slot]).wait()

    fire(0, jnp.int32(0))
    fire(0, jnp.int32(1))

    def chunk_body(c, cnts):
        pltpu.sync_copy(users_hbm.at[pl.ds(c * 1024, 1024)], uchunk)

        def vec_body(v, cnts):
            c0, c1 = cnts
            uv = uchunk[pl.ds(v * 16, 16)]
            hid = lax.shift_right_logical(uv, 14)
            ival = c * 1024 + v * 16 + lane
            m0 = hid == wid * 2
            m1 = hid == wid * 2 + 1
            plsc.store_compressed(lr.at[pl.ds(c0, 16)], uv, mask=m0)
            plsc.store_compressed(li.at[pl.ds(c0, 16)], ival, mask=m0)
            plsc.store_compressed(lr.at[pl.ds(HOFF + c1, 16)], uv, mask=m1)
            plsc.store_compressed(li.at[pl.ds(HOFF + c1, 16)], ival, mask=m1)
            c0 = jnp.minimum(c0 + jnp.sum(m0.astype(jnp.int32)), CAP)
            c1 = jnp.minimum(c1 + jnp.sum(m1.astype(jnp.int32)), CAP)
            return c0, c1

        return lax.fori_loop(0, 64, vec_body, cnts)

    cnts = lax.fori_loop(0, 16, chunk_body,
                         (jnp.int32(0), jnp.int32(0)))

    for half in (0, 1):
        cnt = cnts[half]
        ngroups = (cnt + 15) // 16
        base = (wid * 2 + half) * HSLAB
        nvalid = jnp.clip(N - base, 0, HSLAB)
        nwaves = (nvalid + WAVE - 1) // WAVE

        if half == 1:
            fire(1, jnp.int32(0))
            fire(1, jnp.int32(1))

        def wave_body(w, carry):
            fire(half, w + 2)
            drain(half, w)
            s0 = wstart(half, w)
            slotv = jnp.full((16,), w % NSLOT, dtype=jnp.int32)

            def cgrp(g, wcnt):
                rv = lr[pl.ds(half * HOFF + g * 16, 16)]
                pos = g * 16 + lane
                m = (pos < cnt) & (rv >= s0) & (rv < s0 + WAVE)
                plsc.store_compressed(wl_r.at[pl.ds(wcnt, 16)], rv - s0,
                                      mask=m)
                plsc.store_compressed(wl_p.at[pl.ds(wcnt, 16)], pos, mask=m)
                return wcnt + jnp.sum(m.astype(jnp.int32))

            wcnt = lax.fori_loop(0, ngroups, cgrp, jnp.int32(0))

            def egrp(h, carry):
                roff = wl_r[pl.ds(h * 16, 16)]
                p = wl_p[pl.ds(h * 16, 16)]
                am = (h * 16 + lane) < wcnt
                for k in range(K):
                    kf = jnp.full((16,), k, dtype=jnp.int32)
                    vals = plsc.load_gather(wave_buf, [slotv, kf, roff],
                                            mask=am)
                    plsc.store_scatter(extr, [p, kf], vals, mask=am)
                return carry

            lax.fori_loop(0, (wcnt + 15) // 16, egrp, 0)
            return carry

        lax.fori_loop(0, nwaves, wave_body, 0)
        drain(half, nwaves)
        drain(half, nwaves + 1)

        for t in range(NGRP):
            for u in range(8):
                pos = t * 128 + u * 16 + lane
                vals = li[pl.ds(half * HOFF + t * 128 + u * 16, 16)]
                sink = SINK + u * 16 + lane
                li2d[t, pl.ds(u * 16, 16)] = jnp.where(pos < cnt, vals, sink)
        handles = []
        for t in range(NGRP):
            handles.append(
                pltpu.async_copy(extr.at[pl.ds(t * 128, 128)],
                                 ug_hbm.at[li2d.at[t]], sem_s))
        for h in handles:
            h.wait()


@jax.jit
def _stage_u(users, Ut):
    mesh = plsc.VectorSubcoreMesh(core_axis_name="c", subcore_axis_name="s")
    f = functools.partial(
        pl.kernel,
        mesh=mesh,
        out_type=jax.ShapeDtypeStruct((UG_ROWS, CHUNK), jnp.float32),
        scratch_types=[
            pltpu.VMEM((1024,), jnp.int32),
            pltpu.VMEM((2 * HOFF,), jnp.int32),
            pltpu.VMEM((2 * HOFF,), jnp.int32),
            pltpu.VMEM((96,), jnp.int32),
            pltpu.VMEM((96,), jnp.int32),
            pltpu.VMEM((CAP, CHUNK), jnp.float32),
            pltpu.VMEM((NGRP, CHUNK), jnp.int32),
            pltpu.VMEM((NSLOT, K, WAVE), jnp.float32),
            pltpu.SemaphoreType.DMA,
            pltpu.SemaphoreType.DMA,
            pltpu.SemaphoreType.DMA,
            pltpu.SemaphoreType.DMA,
        ],
        compiler_params=pltpu.CompilerParams(
            needs_layout_passes=False, use_tc_tiling_on_sc=True
        ),
    )(_stream_kernel)
    return f(users, Ut)


def _dot_kernel(users_hbm, jokes_hbm, ug_hbm, Vt_hbm, a_hbm, b_hbm, g_hbm,
                out_hbm,
                idx_u, idx_j, u_buf, v_buf, a_v, b_v, g_v, out_v,
                *sems):
    wid = lax.axis_index("s") * NC + lax.axis_index("c")
    base = wid * BPW
    lane = _iota16()

    for j in range(NCHUNK):
        pltpu.sync_copy(users_hbm.at[pl.ds(base + j * CHUNK, CHUNK)],
                        idx_u.at[j])
        pltpu.sync_copy(jokes_hbm.at[pl.ds(base + j * CHUNK, CHUNK)],
                        idx_j.at[j])
    pltpu.sync_copy(g_hbm, g_v)

    ab_handles = []
    for j in range(NCHUNK):
        ab_handles.append(
            pltpu.async_copy(a_hbm.at[idx_u.at[j]], a_v.at[j], sems[NCHUNK]))
        ab_handles.append(
            pltpu.async_copy(b_hbm.at[idx_j.at[j]], b_v.at[j], sems[NCHUNK]))

    u_handles = {}

    def fire(j):
        u_handles[j] = pltpu.async_copy(
            ug_hbm.at[pl.ds(base + j * CHUNK, CHUNK)], u_buf.at[j], sems[j])

        def body(k, carry):
            pltpu.async_copy(Vt_hbm.at[k].at[idx_j.at[j]],
                             v_buf.at[j, k], sems[j])
            return carry
        lax.fori_loop(0, K, body, 0)

    def drain(j):
        u_handles[j].wait()

        def body(k, carry):
            pltpu.make_async_copy(Vt_hbm.at[k].at[idx_j.at[j]],
                                  v_buf.at[j, k], sems[j]).wait()
            return carry
        lax.fori_loop(0, K, body, 0)

    fire(0)
    fire(1)
    for h in ab_handles:
        h.wait()

    gvec = g_v[...]

    for j in range(NCHUNK):
        drain(j)
        jf = jnp.full((16,), j, dtype=jnp.int32)

        def group_body(grp, carry):
            s = grp * 16
            row = lane + s
            acc = jnp.zeros((16,), dtype=jnp.float32)
            for k in range(K):
                kf = jnp.full((16,), k, dtype=jnp.int32)
                uk = plsc.load_gather(u_buf, [jf, row, kf])
                acc = acc + uk * v_buf[j, k, pl.ds(s, 16)]
            ab = a_v[j, pl.ds(s, 16)] + b_v[j, pl.ds(s, 16)]
            out_v[pl.ds(j * CHUNK + s, 16)] = acc + ab + gvec
            return carry

        lax.fori_loop(0, CHUNK // 16, group_body, 0)
        if j + 2 < NCHUNK:
            fire(j + 2)

    pltpu.sync_copy(out_v, out_hbm.at[pl.ds(base, BPW)])


@jax.jit
def _dot(users, jokes, ug, Vt, a_flat, b_flat, g16):
    mesh = plsc.VectorSubcoreMesh(core_axis_name="c", subcore_axis_name="s")
    f = functools.partial(
        pl.kernel,
        mesh=mesh,
        out_type=jax.ShapeDtypeStruct((B,), jnp.float32),
        scratch_types=[
            pltpu.VMEM((NCHUNK, CHUNK), jnp.int32),
            pltpu.VMEM((NCHUNK, CHUNK), jnp.int32),
            pltpu.VMEM((NCHUNK, CHUNK, CHUNK), jnp.float32),
            pltpu.VMEM((NCHUNK, K, CHUNK), jnp.float32),
            pltpu.VMEM((NCHUNK, CHUNK), jnp.float32),
            pltpu.VMEM((NCHUNK, CHUNK), jnp.float32),
            pltpu.VMEM((16,), jnp.float32),
            pltpu.VMEM((BPW,), jnp.float32),
        ] + [pltpu.SemaphoreType.DMA] * (NCHUNK + 1),
        compiler_params=pltpu.CompilerParams(
            needs_layout_passes=False, use_tc_tiling_on_sc=False
        ),
    )(_dot_kernel)
    return f(users, jokes, ug, Vt, a_flat, b_flat, g16)


def kernel(users, jokes, U, V, a, b, g):
    users = users.astype(jnp.int32)
    jokes = jokes.astype(jnp.int32)
    ug = _stage_u(users, U.T)
    g16 = jnp.broadcast_to(g.astype(jnp.float32), (16,))
    return _dot(users, jokes, ug, V.T, a.reshape(-1), b.reshape(-1), g16)


# speedup vs baseline: 1.1410x; 1.0026x over previous
import functools

import jax
import jax.numpy as jnp
from jax import lax
from jax.experimental import pallas as pl
from jax.experimental.pallas import tpu as pltpu
from jax.experimental.pallas import tpu_sc as plsc

B = 16384
N = 1000000
J = 100000
K = 32
NC = 2
NS = 16
NW = NC * NS
BPW = B // NW
CHUNK = 128
NCHUNK = BPW // CHUNK

HSLAB = 16384
WAVE = 768
NSLOT = 3
CAP = 384
NGRP = CAP // CHUNK
HOFF = 512
SINK = B
UG_ROWS = B + CHUNK


def _iota16():
    return jnp.arange(16, dtype=jnp.int32)


def _stream_kernel(users_hbm, Ut_hbm, ug_hbm,
                   uchunk, lr, li, wl_r, wl_p, extr, li2d, wave_buf,
                   sem_a, sem_b, sem_c, sem_s):
    wid = lax.axis_index("s") * NC + lax.axis_index("c")
    lane = _iota16()
    pad_n = ((N + 127) // 128) * 128
    sems = [sem_a, sem_b, sem_c]

    def wstart(half, w):
        base = (wid * 2 + half) * HSLAB
        s0 = jnp.minimum(base + w * WAVE, pad_n - WAVE)
        return pl.multiple_of(s0, 128)

    def fire(half, w):
        s0 = wstart(half, w)
        for slot in range(NSLOT):
            @pl.when((w % NSLOT) == slot)
            def _():
                pltpu.async_copy(Ut_hbm.at[:, pl.ds(s0, WAVE)],
                                 wave_buf.at[slot], sems[slot])

    def drain(half, w):
        s0 = wstart(half, w)
        for slot in range(NSLOT):
            @pl.when((w % NSLOT) == slot)
            def _():
                pltpu.make_async_copy(Ut_hbm.at[:, pl.ds(s0, WAVE)],
                                      wave_buf.at[slot], sems[slot]).wait()

    fire(0, jnp.int32(0))
    fire(0, jnp.int32(1))

    def chunk_body(c, cnts):
        pltpu.sync_copy(users_hbm.at[pl.ds(c * 1024, 1024)], uchunk)

        def vec_body(v, cnts):
            c0, c1 = cnts
            uv = uchunk[pl.ds(v * 16, 16)]
            hid = lax.shift_right_logical(uv, 14)
            ival = c * 1024 + v * 16 + lane
            m0 = hid == wid * 2
            m1 = hid == wid * 2 + 1
            plsc.store_compressed(lr.at[pl.ds(c0, 16)], uv, mask=m0)
            plsc.store_compressed(li.at[pl.ds(c0, 16)], ival, mask=m0)
            plsc.store_compressed(lr.at[pl.ds(HOFF + c1, 16)], uv, mask=m1)
            plsc.store_compressed(li.at[pl.ds(HOFF + c1, 16)], ival, mask=m1)
            c0 = jnp.minimum(c0 + jnp.sum(m0.astype(jnp.int32)), CAP)
            c1 = jnp.minimum(c1 + jnp.sum(m1.astype(jnp.int32)), CAP)
            return c0, c1

        return lax.fori_loop(0, 64, vec_body, cnts)

    cnts = lax.fori_loop(0, 16, chunk_body,
                         (jnp.int32(0), jnp.int32(0)))

    for half in (0, 1):
        cnt = cnts[half]
        ngroups = (cnt + 15) // 16
        base = (wid * 2 + half) * HSLAB
        nvalid = jnp.clip(N - base, 0, HSLAB)
        nwaves = (nvalid + WAVE - 1) // WAVE

        if half == 1:
            fire(1, jnp.int32(0))
            fire(1, jnp.int32(1))

        def wave_body(w, carry):
            fire(half, w + 2)
            drain(half, w)
            s0 = wstart(half, w)
            slotv = jnp.full((16,), w % NSLOT, dtype=jnp.int32)

            def cgrp(g, wcnt):
                rv = lr[pl.ds(half * HOFF + g * 16, 16)]
                pos = g * 16 + lane
                m = (pos < cnt) & (rv >= s0) & (rv < s0 + WAVE)
                plsc.store_compressed(wl_r.at[pl.ds(wcnt, 16)], rv - s0,
                                      mask=m)
                plsc.store_compressed(wl_p.at[pl.ds(wcnt, 16)], pos, mask=m)
                return wcnt + jnp.sum(m.astype(jnp.int32))

            wcnt = lax.fori_loop(0, ngroups, cgrp, jnp.int32(0))

            def egrp(h, carry):
                roff = wl_r[pl.ds(h * 16, 16)]
                p = wl_p[pl.ds(h * 16, 16)]
                am = (h * 16 + lane) < wcnt
                for k in range(K):
                    kf = jnp.full((16,), k, dtype=jnp.int32)
                    vals = plsc.load_gather(wave_buf, [slotv, kf, roff],
                                            mask=am)
                    plsc.store_scatter(extr, [p, kf], vals, mask=am)
                return carry

            lax.fori_loop(0, (wcnt + 15) // 16, egrp, 0)
            return carry

        lax.fori_loop(0, nwaves, wave_body, 0)
        drain(half, nwaves)
        drain(half, nwaves + 1)

        for t in range(NGRP):
            for u in range(8):
                pos = t * 128 + u * 16 + lane
                vals = li[pl.ds(half * HOFF + t * 128 + u * 16, 16)]
                sink = SINK + u * 16 + lane
                li2d[t, pl.ds(u * 16, 16)] = jnp.where(pos < cnt, vals, sink)
        handles = []
        for t in range(NGRP):
            handles.append(
                pltpu.async_copy(extr.at[pl.ds(t * 128, 128)],
                                 ug_hbm.at[li2d.at[t]], sem_s))
        for h in handles:
            h.wait()


@jax.jit
def _stage_u(users, Ut):
    mesh = plsc.VectorSubcoreMesh(core_axis_name="c", subcore_axis_name="s")
    f = functools.partial(
        pl.kernel,
        mesh=mesh,
        out_type=jax.ShapeDtypeStruct((UG_ROWS, CHUNK), jnp.float32),
        scratch_types=[
            pltpu.VMEM((1024,), jnp.int32),
            pltpu.VMEM((2 * HOFF,), jnp.int32),
            pltpu.VMEM((2 * HOFF,), jnp.int32),
            pltpu.VMEM((96,), jnp.int32),
            pltpu.VMEM((96,), jnp.int32),
            pltpu.VMEM((CAP, CHUNK), jnp.float32),
            pltpu.VMEM((NGRP, CHUNK), jnp.int32),
            pltpu.VMEM((NSLOT, K, WAVE), jnp.float32),
            pltpu.SemaphoreType.DMA,
            pltpu.SemaphoreType.DMA,
            pltpu.SemaphoreType.DMA,
            pltpu.SemaphoreType.DMA,
        ],
        compiler_params=pltpu.CompilerParams(
            needs_layout_passes=False, use_tc_tiling_on_sc=True
        ),
    )(_stream_kernel)
    return f(users, Ut)


def _dot_kernel(users_hbm, jokes_hbm, ug_hbm, Vt_hbm, a_hbm, b_hbm, g_hbm,
                out_hbm,
                idx_u, idx_j, u_buf, v_buf, a_v, b_v, g_v, out_v,
                *sems):
    wid = lax.axis_index("s") * NC + lax.axis_index("c")
    base = wid * BPW
    lane = _iota16()

    for j in range(NCHUNK):
        pltpu.sync_copy(users_hbm.at[pl.ds(base + j * CHUNK, CHUNK)],
                        idx_u.at[j])
        pltpu.sync_copy(jokes_hbm.at[pl.ds(base + j * CHUNK, CHUNK)],
                        idx_j.at[j])
    pltpu.sync_copy(g_hbm, g_v)

    ab_handles = []
    for j in range(NCHUNK):
        ab_handles.append(
            pltpu.async_copy(a_hbm.at[idx_u.at[j]], a_v.at[j], sems[NCHUNK]))
        ab_handles.append(
            pltpu.async_copy(b_hbm.at[idx_j.at[j]], b_v.at[j], sems[NCHUNK]))

    u_handles = {}

    def fire(j):
        u_handles[j] = pltpu.async_copy(
            ug_hbm.at[pl.ds(base + j * CHUNK, CHUNK)], u_buf.at[j], sems[j])

        def body(k, carry):
            pltpu.async_copy(Vt_hbm.at[k].at[idx_j.at[j]],
                             v_buf.at[j, k], sems[j])
            return carry
        lax.fori_loop(0, K, body, 0)

    def drain(j):
        u_handles[j].wait()

        def body(k, carry):
            pltpu.make_async_copy(Vt_hbm.at[k].at[idx_j.at[j]],
                                  v_buf.at[j, k], sems[j]).wait()
            return carry
        lax.fori_loop(0, K, body, 0)

    fire(0)
    fire(1)
    for h in ab_handles:
        h.wait()

    gvec = g_v[...]

    for j in range(NCHUNK):
        drain(j)
        jf = jnp.full((16,), j, dtype=jnp.int32)

        def group_body(grp, carry):
            s = grp * 16
            row = lane + s
            acc = jnp.zeros((16,), dtype=jnp.float32)
            for k in range(K):
                kf = jnp.full((16,), k, dtype=jnp.int32)
                uk = plsc.load_gather(u_buf, [jf, row, kf])
                acc = acc + uk * v_buf[j, k, pl.ds(s, 16)]
            ab = a_v[j, pl.ds(s, 16)] + b_v[j, pl.ds(s, 16)]
            out_v[pl.ds(j * CHUNK + s, 16)] = acc + ab + gvec
            return carry

        lax.fori_loop(0, CHUNK // 16, group_body, 0)
        if j + 2 < NCHUNK:
            fire(j + 2)

    pltpu.sync_copy(out_v, out_hbm.at[pl.ds(base, BPW)])


@jax.jit
def _dot(users, jokes, ug, Vt, a_flat, b_flat, g16):
    mesh = plsc.VectorSubcoreMesh(core_axis_name="c", subcore_axis_name="s")
    f = functools.partial(
        pl.kernel,
        mesh=mesh,
        out_type=jax.ShapeDtypeStruct((B,), jnp.float32),
        scratch_types=[
            pltpu.VMEM((NCHUNK, CHUNK), jnp.int32),
            pltpu.VMEM((NCHUNK, CHUNK), jnp.int32),
            pltpu.VMEM((NCHUNK, CHUNK, CHUNK), jnp.float32),
            pltpu.VMEM((NCHUNK, K, CHUNK), jnp.float32),
            pltpu.VMEM((NCHUNK, CHUNK), jnp.float32),
            pltpu.VMEM((NCHUNK, CHUNK), jnp.float32),
            pltpu.VMEM((16,), jnp.float32),
            pltpu.VMEM((BPW,), jnp.float32),
        ] + [pltpu.SemaphoreType.DMA] * (NCHUNK + 1),
        compiler_params=pltpu.CompilerParams(
            needs_layout_passes=False, use_tc_tiling_on_sc=False
        ),
    )(_dot_kernel)
    return f(users, jokes, ug, Vt, a_flat, b_flat, g16)


def kernel(users, jokes, U, V, a, b, g):
    users = users.astype(jnp.int32)
    jokes = jokes.astype(jnp.int32)
    ug = _stage_u(users, U.T)
    g16 = jnp.broadcast_to(g.astype(jnp.float32), (16,))
    return _dot(users, jokes, ug, V.T, a.reshape(-1), b.reshape(-1), g16)
